# Initial kernel scaffold; baseline (speedup 1.0000x reference)
import jax
import jax.numpy as jnp
from jax.experimental import pallas as pl


def kernel(positions, shifts, W, species, edge_index):
    raise NotImplementedError("write your pallas kernel here")


# speedup vs baseline: 1.0000x; 1.0000x over previous
import jax
import jax.numpy as jnp
import numpy as np

N_RADIAL = 7
N_BASIS = 7
R_CUTOFF = 5.0
R_MIN = 0.5
N_RADIAL_SQ = N_RADIAL * N_RADIAL
N_RADIAL_CB = N_RADIAL_SQ * N_RADIAL

_t2 = np.array([[i, j] for i in range(N_RADIAL) for j in range(i, N_RADIAL)], dtype=np.int32)
_TRIU2 = jnp.asarray(N_RADIAL * _t2[:, 0] + _t2[:, 1])
_t3 = np.array([[i, j, k] for i in range(N_RADIAL) for j in range(i, N_RADIAL) for k in range(j, N_RADIAL)], dtype=np.int32)
_TRIU3 = jnp.asarray(N_RADIAL_SQ * _t3[:, 0] + N_RADIAL * _t3[:, 1] + _t3[:, 2])
_N_SYMM01 = _t2.shape[0] * N_RADIAL


def _cosine_cutoff(r):
    return jnp.where(r < R_CUTOFF, 0.5 * (jnp.cos(jnp.pi * r / R_CUTOFF) + 1.0), 0.0)


def kernel(positions, shifts, W, species, edge_index):
    idx_i = edge_index[0]
    idx_j = edge_index[1]
    r_ij = positions[idx_i] - positions[idx_j] - shifts
    r_len = jnp.linalg.norm(r_ij, axis=-1)
    r_vec = r_ij / (r_len[..., None] + 1e-12)
    centers = jnp.linspace(R_MIN, R_CUTOFF, N_BASIS)
    sigma = (R_CUTOFF - R_MIN) / N_BASIS
    basis = jnp.exp(-((r_len[:, None] - centers[None, :]) ** 2) / (2.0 * sigma * sigma))
    W_ij = W[species[idx_i], species[idx_j]]
    e = _cosine_cutoff(r_len)[:, None] * jnp.einsum('eb,erb->er', basis, W_ij)
    xyz = e[:, :, None] * r_vec[:, None, :]
    x_xyz = xyz * r_vec[:, None, 0:1]
    y_yz = xyz[:, :, 1:] * r_vec[:, None, 1:2]
    x_x_xyz = x_xyz * r_vec[:, None, 0:1]
    xy_y_yz = r_vec[:, None, 0:2, None] * y_yz[:, :, None, :]
    n = species.shape[0]
    seg = lambda t: jax.ops.segment_sum(t, idx_i, num_segments=n)
    e = seg(e)
    xyz = seg(xyz)
    x_xyz = seg(x_xyz)
    y_yz = seg(y_yz)
    x_x_xyz = seg(x_x_xyz)
    xy_y_yz = seg(xy_y_yz)
    z_z = e - x_xyz[:, :, 0] - y_yz[:, :, 0]
    x_z_z = xyz[:, :, 0] - x_x_xyz[:, :, 0] - xy_y_yz[:, :, 0, 0]
    y_z_z = xyz[:, :, 1] - x_x_xyz[:, :, 1] - xy_y_yz[:, :, 1, 0]
    z_z_z = xyz[:, :, 2] - x_x_xyz[:, :, 2] - xy_y_yz[:, :, 1, 1]
    y_xyz = jnp.concatenate([x_xyz[:, :, 1:2], y_yz], -1)
    z_xyz = jnp.concatenate([x_xyz[:, :, 2:3], y_yz[:, :, 1:2], z_z[:, :, None]], -1)
    xyz_xyz = jnp.stack([x_xyz, y_xyz, z_xyz], -2)
    x_y_xyz = jnp.concatenate([x_x_xyz[:, :, 1:2], xy_y_yz[:, :, 0, 0:2]], -1)
    x_z_xyz = jnp.concatenate([x_x_xyz[:, :, 2:3], xy_y_yz[:, :, 0, 1:2], x_z_z[:, :, None]], -1)
    x_xyz_xyz = jnp.stack([x_x_xyz, x_y_xyz, x_z_xyz], -2)
    y_y_xyz = jnp.concatenate([xy_y_yz[:, :, 0, 0:1], xy_y_yz[:, :, 1, :]], -1)
    y_z_xyz = jnp.concatenate([xy_y_yz[:, :, 0, 1:2], xy_y_yz[:, :, 1, 1:2], y_z_z[:, :, None]], -1)
    z_z_xyz = jnp.stack([x_z_z, y_z_z, z_z_z], -1)
    y_xyz_xyz = jnp.stack([x_y_xyz, y_y_xyz, y_z_xyz], -2)
    z_xyz_xyz = jnp.stack([x_z_xyz, y_z_xyz, z_z_xyz], -2)
    xyz_xyz_xyz = jnp.stack([x_xyz_xyz, y_xyz_xyz, z_xyz_xyz], -3)
    m0 = e
    m1 = xyz
    m2 = xyz_xyz
    m3 = xyz_xyz_xyz
    contr_0 = m0
    contr_1 = jnp.einsum('ari,asi->ars', m1, m1).reshape(-1, N_RADIAL_SQ)
    contr_2 = jnp.einsum('arij,asij->ars', m2, m2).reshape(-1, N_RADIAL_SQ)
    contr_3 = jnp.einsum('arijk,asijk->ars', m3, m3).reshape(-1, N_RADIAL_SQ)
    contr_4 = jnp.einsum('arij,asik,atjk->arst', m2, m2, m2).reshape(-1, N_RADIAL_CB)
    contr_5 = jnp.einsum('ari,asj,atij->arst', m1, m1, m2).reshape(-1, N_RADIAL_SQ, N_RADIAL)
    contr_6 = jnp.einsum('arijk,asijl,atkl->arst', m3, m3, m2).reshape(-1, N_RADIAL_SQ, N_RADIAL)
    contr_7 = jnp.einsum('arijk,asij,atk->arst', m3, m2, m1)
    gm = [contr_0,
          jnp.take(contr_1, _TRIU2, axis=1),
          jnp.take(contr_2, _TRIU2, axis=1),
          jnp.take(contr_3, _TRIU2, axis=1),
          jnp.take(contr_4, _TRIU3, axis=1),
          jnp.take(contr_5, _TRIU2, axis=1).reshape(-1, _N_SYMM01),
          jnp.take(contr_6, _TRIU2, axis=1).reshape(-1, _N_SYMM01),
          contr_7.reshape(-1, N_RADIAL_CB)]
    return jnp.concatenate(gm, -1)


# speedup vs baseline: 16.2618x; 16.2616x over previous
import functools

import jax
import jax.numpy as jnp
import numpy as np
from jax.experimental import pallas as pl

N_RADIAL = 7
N_BASIS = 7
R_CUTOFF = 5.0
R_MIN = 0.5

N_MONO = 16
N_COMP = N_RADIAL * N_MONO
N_OUT = 910
N_OUT_PAD = 912
BLK = 1024

P2 = [(i, j) for i in range(N_RADIAL) for j in range(i, N_RADIAL)]
T3 = [(i, j, k) for i in range(N_RADIAL) for j in range(i, N_RADIAL)
      for k in range(j, N_RADIAL)]

_PAIR6 = {(0, 0): 0, (0, 1): 1, (0, 2): 2, (1, 1): 3, (1, 2): 4, (2, 2): 5}
W2 = [1.0, 2.0, 2.0, 1.0, 2.0, 1.0]
_TRIP10 = {(0, 0, 0): 0, (0, 0, 1): 1, (0, 0, 2): 2, (0, 1, 1): 3,
           (0, 1, 2): 4, (0, 2, 2): 5, (1, 1, 1): 6, (1, 1, 2): 7,
           (1, 2, 2): 8, (2, 2, 2): 9}
W3 = [1.0, 3.0, 3.0, 3.0, 6.0, 3.0, 1.0, 3.0, 3.0, 1.0]


def _p6(i, j):
    return _PAIR6[(i, j) if i <= j else (j, i)]


def _t10(i, j, k):
    return _TRIP10[tuple(sorted((i, j, k)))]


def _contract_body(m_ref, o_ref):
    g = lambda r, s: m_ref[r * N_MONO + s, :]
    R = range(N_RADIAL)
    e = [g(r, 0) for r in R]
    m1 = [[g(r, 1 + i) for i in range(3)] for r in R]
    m2 = [[g(r, 4), g(r, 5), g(r, 6), g(r, 7), g(r, 8),
           e[r] - g(r, 4) - g(r, 7)] for r in R]
    m3 = []
    for r in R:
        xxx, xxy, xxz = g(r, 9), g(r, 10), g(r, 11)
        xyy, xyz = g(r, 12), g(r, 13)
        yyy, yyz = g(r, 14), g(r, 15)
        xzz = m1[r][0] - xxx - xyy
        yzz = m1[r][1] - xxy - yyy
        zzz = m1[r][2] - xxz - yyz
        m3.append({(0, 0, 0): xxx, (0, 0, 1): xxy, (0, 0, 2): xxz,
                   (0, 1, 1): xyy, (0, 1, 2): xyz, (0, 2, 2): xzz,
                   (1, 1, 1): yyy, (1, 1, 2): yyz, (1, 2, 2): yzz,
                   (2, 2, 2): zzz})
    m2f = lambda r, i, j: m2[r][_p6(i, j)]
    m3f = lambda r, i, j, k: m3[r][tuple(sorted((i, j, k)))]
    m2w = [[W2[u] * m2[r][u] for u in range(6)] for r in R]
    PAIRS6 = [(0, 0), (0, 1), (0, 2), (1, 1), (1, 2), (2, 2)]
    m3uk = [[[m3f(r, i, j, k) for k in range(3)] for (i, j) in PAIRS6]
            for r in R]
    m3pw = [[[W2[u] * m3uk[r][u][k] for k in range(3)] for u in range(6)]
            for r in R]
    m3w = [{t: W3[_TRIP10[t]] * m3[r][t] for t in _TRIP10} for r in R]

    row = [0]

    def emit(v):
        o_ref[row[0], :] = v
        row[0] += 1

    for r in R:
        emit(e[r])
    for (r, s) in P2:
        acc = m1[r][0] * m1[s][0]
        for i in (1, 2):
            acc += m1[r][i] * m1[s][i]
        emit(acc)
    for (r, s) in P2:
        acc = m2w[r][0] * m2[s][0]
        for u in range(1, 6):
            acc += m2w[r][u] * m2[s][u]
        emit(acc)
    for (r, s) in P2:
        keys = list(_TRIP10)
        acc = m3w[r][keys[0]] * m3[s][keys[0]]
        for t in keys[1:]:
            acc += m3w[r][t] * m3[s][t]
        emit(acc)
    a4 = {}
    for (r, s) in P2:
        A = [[None] * 3 for _ in range(3)]
        for j in range(3):
            for k in range(3):
                acc = m2f(r, 0, j) * m2f(s, 0, k)
                for i in (1, 2):
                    acc += m2f(r, i, j) * m2f(s, i, k)
                A[j][k] = acc
        a4[(r, s)] = A
    for (r, s, t) in T3:
        A = a4[(r, s)]
        acc = None
        for j in range(3):
            for k in range(3):
                term = A[j][k] * m2f(t, j, k)
                acc = term if acc is None else acc + term
        emit(acc)
    for (r, s) in P2:
        for t in R:
            acc = None
            for i in range(3):
                gi = m1[s][0] * m2f(t, i, 0)
                for j in (1, 2):
                    gi += m1[s][j] * m2f(t, i, j)
                term = m1[r][i] * gi
                acc = term if acc is None else acc + term
            emit(acc)
    for (r, s) in P2:
        A = [[None] * 3 for _ in range(3)]
        for k in range(3):
            for ll in range(3):
                acc = m3pw[r][0][k] * m3uk[s][0][ll]
                for u in range(1, 6):
                    acc += m3pw[r][u][k] * m3uk[s][u][ll]
                A[k][ll] = acc
        for t in R:
            acc = None
            for k in range(3):
                for ll in range(3):
                    term = A[k][ll] * m2f(t, k, ll)
                    acc = term if acc is None else acc + term
            emit(acc)
    for r in R:
        for s in R:
            B = [None] * 3
            for k in range(3):
                acc = m3uk[r][0][k] * m2w[s][0]
                for u in range(1, 6):
                    acc += m3uk[r][u][k] * m2w[s][u]
                B[k] = acc
            for t in R:
                acc = B[0] * m1[t][0]
                for k in (1, 2):
                    acc += B[k] * m1[t][k]
                emit(acc)
    assert row[0] == N_OUT
    o_ref[N_OUT, :] = e[0] * 0.0
    o_ref[N_OUT + 1, :] = e[0] * 0.0


def _contract_tc(m_t):
    np_ = m_t.shape[1]
    nblk = np_ // BLK
    return pl.pallas_call(
        _contract_body,
        grid=(nblk,),
        in_specs=[pl.BlockSpec((N_COMP, BLK), lambda i: (0, i))],
        out_specs=pl.BlockSpec((N_OUT_PAD, BLK), lambda i: (0, i)),
        out_shape=jax.ShapeDtypeStruct((N_OUT_PAD, np_), jnp.float32),
    )(m_t)


def _cosine_cutoff(r):
    return jnp.where(r < R_CUTOFF, 0.5 * (jnp.cos(jnp.pi * r / R_CUTOFF) + 1.0), 0.0)


def _edge_stage(positions, shifts, W, species, edge_index):
    idx_i = edge_index[0]
    idx_j = edge_index[1]
    r_ij = positions[idx_i] - positions[idx_j] - shifts
    r_len = jnp.linalg.norm(r_ij, axis=-1)
    v = r_ij / (r_len[..., None] + 1e-12)
    centers = jnp.linspace(R_MIN, R_CUTOFF, N_BASIS)
    sigma = (R_CUTOFF - R_MIN) / N_BASIS
    basis = jnp.exp(-((r_len[:, None] - centers[None, :]) ** 2) / (2.0 * sigma * sigma))
    W_ij = W[species[idx_i], species[idx_j]]
    e = _cosine_cutoff(r_len)[:, None] * jnp.einsum('eb,erb->er', basis, W_ij)
    x, y, z = v[:, 0], v[:, 1], v[:, 2]
    mono = jnp.stack([jnp.ones_like(x), x, y, z,
                      x * x, x * y, x * z, y * y, y * z,
                      x * x * x, x * x * y, x * x * z, x * y * y, x * y * z,
                      y * y * y, y * y * z], axis=-1)
    ev = (e[:, :, None] * mono[:, None, :]).reshape(-1, N_COMP)
    n = species.shape[0]
    return jax.ops.segment_sum(ev, idx_i, num_segments=n)


def kernel(positions, shifts, W, species, edge_index):
    n = species.shape[0]
    m = _edge_stage(positions, shifts, W, species, edge_index)
    n_pad = ((n + BLK - 1) // BLK) * BLK
    m_t = jnp.zeros((N_COMP, n_pad), jnp.float32).at[:, :n].set(m.T)
    out_t = _contract_tc(m_t)
    return out_t[:N_OUT, :n].T
